# baseline (speedup 1.0000x reference)
import functools

import jax
import jax.numpy as jnp
from jax import lax
from jax.experimental import pallas as pl
from jax.experimental.pallas import tpu as pltpu
from jax.experimental.pallas import tpu_sc as plsc

B = 16384
F = 26
V = 100000
D = 16

NC = 2
NS = 16
NW = NC * NS
P = F * D
PPW = P // NW
S0 = 49920
S1 = V - S0
ICH = 2048


def _fd(p):
    f = p // D
    return f, p - f * D


def _body(idx_hbm, tab_hbm, out_hbm, buf0, buf1, ibuf, obuf, sem0, sem1, osem):
    cid = lax.axis_index("c")
    sid = lax.axis_index("s")
    w = sid * NC + cid
    p0 = w * PPW
    lane = lax.iota(jnp.int32, 16)

    f0, d0 = _fd(p0)
    pltpu.async_copy(tab_hbm.at[f0, d0, pl.ds(0, S0)], buf0, sem0)

    @pl.loop(0, PPW)
    def _plane(i):
        p = p0 + i
        f, d = _fd(p)

        pltpu.make_async_copy(tab_hbm.at[f, d, pl.ds(0, S0)], buf0, sem0).wait()
        pltpu.async_copy(tab_hbm.at[f, d, pl.ds(S0, S1)], buf1, sem1)

        @pl.when(i > 0)
        def _drain_out():
            pltpu.make_async_copy(obuf, out_hbm.at[f, d], osem).wait()

        @pl.loop(0, B // ICH)
        def _pass_a(c):
            pltpu.sync_copy(idx_hbm.at[f, pl.ds(c * ICH, ICH)], ibuf)
            for j in range(ICH // 16):
                ids = ibuf[pl.ds(j * 16, 16)]
                vals = plsc.load_gather(buf0, [jnp.minimum(ids, S0 - 1)])
                obuf[pl.ds(c * ICH + j * 16, 16)] = vals

        pltpu.make_async_copy(tab_hbm.at[f, d, pl.ds(S0, S1)], buf1, sem1).wait()

        @pl.when(i < PPW - 1)
        def _prefetch_next():
            fn, dn = _fd(p + 1)
            pltpu.async_copy(tab_hbm.at[fn, dn, pl.ds(0, S0)], buf0, sem0)

        @pl.loop(0, B // ICH)
        def _pass_b(c):
            pltpu.sync_copy(idx_hbm.at[f, pl.ds(c * ICH, ICH)], ibuf)
            for j in range(ICH // 16):
                base = c * ICH + j * 16
                ids = ibuf[pl.ds(j * 16, 16)]
                vals = plsc.load_gather(buf1, [jnp.maximum(ids - S0, 0)])
                plsc.store_scatter(obuf, [base + lane], vals, mask=ids >= S0)

        pltpu.async_copy(obuf, out_hbm.at[f, d], osem)

    ff, df = _fd(p0 + PPW - 1)
    pltpu.make_async_copy(obuf, out_hbm.at[ff, df], osem).wait()


_lookup = functools.partial(
    pl.kernel,
    out_type=jax.ShapeDtypeStruct((F, D, B), jnp.float32),
    mesh=plsc.VectorSubcoreMesh(
        core_axis_name="c", subcore_axis_name="s", num_cores=NC, num_subcores=NS
    ),
    scratch_types=[
        pltpu.VMEM((S0,), jnp.float32),
        pltpu.VMEM((S1,), jnp.float32),
        pltpu.VMEM((ICH,), jnp.int32),
        pltpu.VMEM((B,), jnp.float32),
        pltpu.SemaphoreType.DMA,
        pltpu.SemaphoreType.DMA,
        pltpu.SemaphoreType.DMA,
    ],
    compiler_params=pltpu.CompilerParams(
        use_tc_tiling_on_sc=True, needs_layout_passes=False
    ),
)(_body)


@jax.jit
def kernel(indices, tables):
    idx_t = indices.T
    tab_t = tables.transpose(0, 2, 1)
    out_t = _lookup(idx_t, tab_t)
    return out_t.transpose(2, 0, 1)


# speedup vs baseline: 2.4209x; 2.4209x over previous
import functools

import jax
import jax.numpy as jnp
from jax import lax
from jax.experimental import pallas as pl
from jax.experimental.pallas import tpu as pltpu
from jax.experimental.pallas import tpu_sc as plsc

B = 16384
F = 26
V = 100000
D = 16

NC = 2
NS = 16
NW = NC * NS
P = F * D
PPW = P // NW
CH = 2048
NCH = B // CH


def _fd(p):
    f = p // D
    return f, p - f * D


def _body(idx_hbm, tab_hbm, out_hbm, idx_v, plane_v, obuf, sem_a, sem_b):
    cid = lax.axis_index("c")
    sid = lax.axis_index("s")
    w = sid * NC + cid
    p0 = w * PPW
    sems = (sem_a, sem_b)

    @pl.loop(0, PPW)
    def _plane(i):
        p = p0 + i
        f, d = _fd(p)

        @pl.when(jnp.logical_or(i == 0, d == 0))
        def _load_idx():
            pltpu.sync_copy(idx_hbm.at[f], idx_v)

        pltpu.sync_copy(tab_hbm.at[f, d], plane_v)

        @pl.loop(0, NCH, step=2)
        def _chunks(cc):
            for sub in range(2):
                c = cc + sub

                @pl.when(jnp.logical_or(i > 0, cc > 0))
                def _drain():
                    pltpu.make_async_copy(
                        obuf.at[sub],
                        out_hbm.at[f, d, pl.ds(c * CH, CH)],
                        sems[sub],
                    ).wait()

                for j in range(CH // 16):
                    ids = idx_v[pl.ds(c * CH + j * 16, 16)]
                    obuf[sub, pl.ds(j * 16, 16)] = plsc.load_gather(
                        plane_v, [ids]
                    )
                pltpu.async_copy(
                    obuf.at[sub],
                    out_hbm.at[f, d, pl.ds(c * CH, CH)],
                    sems[sub],
                )

    ff, df = _fd(p0 + PPW - 1)
    for sub in range(2):
        pltpu.make_async_copy(
            obuf.at[sub],
            out_hbm.at[ff, df, pl.ds((NCH - 2 + sub) * CH, CH)],
            sems[sub],
        ).wait()


_lookup = functools.partial(
    pl.kernel,
    out_type=jax.ShapeDtypeStruct((F, D, B), jnp.float32),
    mesh=plsc.VectorSubcoreMesh(
        core_axis_name="c", subcore_axis_name="s", num_cores=NC, num_subcores=NS
    ),
    scratch_types=[
        pltpu.VMEM((B,), jnp.int32),
        pltpu.VMEM((V,), jnp.float32),
        pltpu.VMEM((2, CH), jnp.float32),
        pltpu.SemaphoreType.DMA,
        pltpu.SemaphoreType.DMA,
    ],
    compiler_params=pltpu.CompilerParams(
        use_tc_tiling_on_sc=True, needs_layout_passes=False
    ),
)(_body)


@jax.jit
def kernel(indices, tables):
    idx_t = indices.T
    tab_t = tables.transpose(0, 2, 1)
    out_t = _lookup(idx_t, tab_t)
    return out_t.transpose(2, 0, 1)


# speedup vs baseline: 4.3737x; 1.8066x over previous
import functools

import jax
import jax.numpy as jnp
from jax import lax
from jax.experimental import pallas as pl
from jax.experimental.pallas import tpu as pltpu
from jax.experimental.pallas import tpu_sc as plsc

B = 16384
F = 26
V = 100000
D = 16

NC = 2
NS = 16
NW = NC * NS
P = F * D
PPW = P // NW
CH = 2048
NCH = B // CH


def _fd(p):
    f = p // D
    return f, p - f * D


def _body(idx_hbm, tab_hbm, out_hbm, idx_v, plane_v, obuf, sem_a, sem_b):
    cid = lax.axis_index("c")
    sid = lax.axis_index("s")
    w = sid * NC + cid
    p0 = w * PPW
    sems = (sem_a, sem_b)

    @pl.loop(0, PPW)
    def _plane(i):
        p = p0 + i
        f, d = _fd(p)

        @pl.when(jnp.logical_or(i == 0, d == 0))
        def _load_idx():
            pltpu.sync_copy(idx_hbm.at[f], idx_v)

        pltpu.sync_copy(tab_hbm.at[f, d], plane_v)

        @pl.loop(0, NCH, step=2)
        def _chunks(cc):
            for sub in range(2):
                c = cc + sub

                @pl.when(jnp.logical_or(i > 0, cc > 0))
                def _drain():
                    pltpu.make_async_copy(
                        obuf.at[sub],
                        out_hbm.at[f, d, pl.ds(c * CH, CH)],
                        sems[sub],
                    ).wait()

                for j0 in range(0, CH // 16, 8):
                    ids8 = [
                        idx_v[pl.ds(c * CH + (j0 + k) * 16, 16)]
                        for k in range(8)
                    ]
                    vals8 = [plsc.load_gather(plane_v, [ids]) for ids in ids8]
                    for k in range(8):
                        obuf[sub, pl.ds((j0 + k) * 16, 16)] = vals8[k]
                pltpu.async_copy(
                    obuf.at[sub],
                    out_hbm.at[f, d, pl.ds(c * CH, CH)],
                    sems[sub],
                )

    ff, df = _fd(p0 + PPW - 1)
    for sub in range(2):
        pltpu.make_async_copy(
            obuf.at[sub],
            out_hbm.at[ff, df, pl.ds((NCH - 2 + sub) * CH, CH)],
            sems[sub],
        ).wait()


_lookup = functools.partial(
    pl.kernel,
    out_type=jax.ShapeDtypeStruct((F, D, B), jnp.float32),
    mesh=plsc.VectorSubcoreMesh(
        core_axis_name="c", subcore_axis_name="s", num_cores=NC, num_subcores=NS
    ),
    scratch_types=[
        pltpu.VMEM((B,), jnp.int32),
        pltpu.VMEM((V,), jnp.float32),
        pltpu.VMEM((2, CH), jnp.float32),
        pltpu.SemaphoreType.DMA,
        pltpu.SemaphoreType.DMA,
    ],
    compiler_params=pltpu.CompilerParams(
        use_tc_tiling_on_sc=True, needs_layout_passes=False
    ),
)(_body)


@jax.jit
def kernel(indices, tables):
    idx_t = indices.T
    tab_t = tables.transpose(0, 2, 1)
    out_t = _lookup(idx_t, tab_t)
    return out_t.transpose(2, 0, 1)
